# baseline (speedup 1.0000x reference)
import functools

import jax
import jax.numpy as jnp
from jax import lax
from jax.experimental import pallas as pl
from jax.experimental.pallas import tpu as pltpu
from jax.experimental.pallas import tpu_sc as plsc

N = 10000
E = 320000
D = 128
D2 = D // 2
NEG_SLOPE = 0.01

NC = 2
NS = 16
L = 16
NW = NC * NS
K = 128
EPT = E // NS
NCHUNK = -(-EPT // K)
EPTP = NCHUNK * K
NP = 10240
RPT = NP // NS
ZB = 128


def _sc_mesh():
    return plsc.VectorSubcoreMesh(
        core_axis_name="c", subcore_axis_name="s", num_cores=NC, num_subcores=NS
    )


EPW = E // NW


@functools.cache
def _get_sc_bincount():
    @functools.partial(
        pl.kernel,
        out_type=(
            jax.ShapeDtypeStruct((NW, NP), jnp.float32),
            jax.ShapeDtypeStruct((NW, NP), jnp.float32),
        ),
        mesh=_sc_mesh(),
        compiler_params=pltpu.CompilerParams(needs_layout_passes=False),
        scratch_types=[
            pltpu.VMEM((EPW,), jnp.int32),
            pltpu.VMEM((EPW,), jnp.int32),
            pltpu.VMEM((NP,), jnp.float32),
            pltpu.VMEM((NP,), jnp.float32),
        ],
    )
    def _sc_bincount(src_h, dst_h, osrc_h, odst_h, sall, dall, csrc, cdst):
        c = lax.axis_index("c")
        s = lax.axis_index("s")
        wid = c * NS + s

        ebase = wid * EPW
        pltpu.sync_copy(src_h.at[pl.ds(ebase, EPW)], sall)
        pltpu.sync_copy(dst_h.at[pl.ds(ebase, EPW)], dall)

        def zero(i, _):
            z = jnp.zeros((L,), jnp.float32)
            csrc[pl.ds(i * L, L)] = z
            cdst[pl.ds(i * L, L)] = z
            return 0

        lax.fori_loop(0, NP // L, zero, 0)

        def grp(g, _):
            sv = sall[pl.ds(g * L, L)]
            cnt, last = plsc.scan_count(sv)
            plsc.addupdate_scatter(csrc, [sv], cnt.astype(jnp.float32), mask=last)
            dv = dall[pl.ds(g * L, L)]
            cnt2, last2 = plsc.scan_count(dv)
            plsc.addupdate_scatter(cdst, [dv], cnt2.astype(jnp.float32), mask=last2)
            return 0

        lax.fori_loop(0, EPW // L, grp, 0)

        pltpu.sync_copy(csrc, osrc_h.at[wid])
        pltpu.sync_copy(cdst, odst_h.at[wid])

    return _sc_bincount


@functools.cache
def _get_sc_scatter():
    @functools.partial(
        pl.kernel,
        out_type=jax.ShapeDtypeStruct((NC, NP, D2), jnp.float32),
        mesh=_sc_mesh(),
        compiler_params=pltpu.CompilerParams(
            needs_layout_passes=False, use_tc_tiling_on_sc=False
        ),
        scratch_types=[
            pltpu.VMEM((NCHUNK, K), jnp.int32),
            pltpu.VMEM((NCHUNK, K), jnp.int32),
            pltpu.VMEM((EPTP,), jnp.float32),
            pltpu.VMEM((2, K, D2), jnp.float32),
            pltpu.VMEM((ZB, D2), jnp.float32),
            pltpu.VMEM_SHARED((NP, D2), jnp.float32),
            pltpu.SemaphoreType.DMA((2,)),
        ],
    )
    def _sc_scatter(src_h, dst_h, ew_h, u_h, out_h,
                    sidx, didx, ewv, rows, zer, agg, sem):
        c = lax.axis_index("c")
        s = lax.axis_index("s")

        pltpu.sync_copy(src_h.at[s], sidx)
        pltpu.sync_copy(dst_h.at[s], didx)
        pltpu.sync_copy(ew_h.at[s], ewv)

        def fill_zer(i, _):
            for f in range(D2 // L):
                zer[i, pl.ds(f * L, L)] = jnp.zeros((L,), jnp.float32)
            return 0

        lax.fori_loop(0, ZB, fill_zer, 0)

        r0 = s * RPT
        for j in range(RPT // ZB):
            pltpu.sync_copy(zer, agg.at[pl.ds(r0 + j * ZB, ZB)])
        plsc.subcore_barrier()

        uc = u_h.at[c]

        def start_gather(i):
            b = lax.rem(i, 2)
            pltpu.async_copy(uc.at[sidx.at[i]], rows.at[b], sem.at[b])

        def wait_gather(i):
            b = lax.rem(i, 2)
            pltpu.make_async_copy(uc.at[sidx.at[i]], rows.at[b], sem.at[b]).wait()

        start_gather(0)

        def chunk(i, _):
            b = lax.rem(i, 2)
            wait_gather(i)

            @pl.when(i < NCHUNK - 1)
            def _():
                start_gather(i + 1)

            def scale(g, _):
                cvec = ewv[pl.ds(i * K + g * L, L)]
                for r in range(L):
                    ce = jnp.full((L,), cvec[r], jnp.float32)
                    e = g * L + r
                    for f in range(D2 // L):
                        rows[b, e, pl.ds(f * L, L)] = rows[b, e, pl.ds(f * L, L)] * ce
                return 0

            lax.fori_loop(0, K // L, scale, 0)
            pltpu.sync_copy(rows.at[b], agg.at[didx.at[i]], add=True)
            return 0

        lax.fori_loop(0, NCHUNK, chunk, 0)
        plsc.subcore_barrier()

        for j in range(RPT // ZB):
            sl = pl.ds(r0 + j * ZB, ZB)
            pltpu.sync_copy(agg.at[sl], out_h.at[c, sl])

    return _sc_scatter


BM = 1000


def _split(y, o_ref):
    o_ref[0] = y[:, :D2]
    o_ref[1] = y[:, D2:]


def _mm1_body(x_ref, ns_ref, w_ref, o_ref):
    y = jnp.dot(x_ref[:] * ns_ref[:], w_ref[:], preferred_element_type=jnp.float32)
    _split(y, o_ref)


def _mm2_body(p_ref, ns_ref, nd_ref, w_ref, o_ref):
    h = jnp.concatenate([p_ref[0], p_ref[1]], axis=1) * nd_ref[:]
    h = jnp.where(h > 0, h, NEG_SLOPE * h)
    y = jnp.dot(h * ns_ref[:], w_ref[:], preferred_element_type=jnp.float32)
    _split(y, o_ref)


def _fin_body(p_ref, nd_ref, o_ref):
    h = jnp.concatenate([p_ref[0], p_ref[1]], axis=1) * nd_ref[:]
    h = jnp.where(h > 0, h, NEG_SLOPE * h)
    part = jnp.sum(h, axis=0, keepdims=True) * (1.0 / N)

    @pl.when(pl.program_id(0) == 0)
    def _():
        o_ref[:] = jnp.zeros_like(o_ref)

    o_ref[:] = o_ref[:] + part


_col_spec = pl.BlockSpec((BM, 1), lambda i: (i, 0))
_p_spec = pl.BlockSpec((NC, BM, D2), lambda i: (0, i, 0))
_u_spec = pl.BlockSpec((NC, BM, D2), lambda i: (0, i, 0))
_w_spec = pl.BlockSpec((D, D), lambda i: (0, 0))

_mm1 = pl.pallas_call(
    _mm1_body,
    grid=(N // BM,),
    in_specs=[pl.BlockSpec((BM, D), lambda i: (i, 0)), _col_spec, _w_spec],
    out_specs=_u_spec,
    out_shape=jax.ShapeDtypeStruct((NC, N, D2), jnp.float32),
)

_mm2 = pl.pallas_call(
    _mm2_body,
    grid=(N // BM,),
    in_specs=[_p_spec, _col_spec, _col_spec, _w_spec],
    out_specs=_u_spec,
    out_shape=jax.ShapeDtypeStruct((NC, N, D2), jnp.float32),
)

_fin = pl.pallas_call(
    _fin_body,
    grid=(N // BM,),
    in_specs=[_p_spec, _col_spec],
    out_specs=pl.BlockSpec((1, D), lambda i: (0, 0)),
    out_shape=jax.ShapeDtypeStruct((1, D), jnp.float32),
)


def kernel(updated_feats, edge_index, edge_weight, W1, W2):
    src = edge_index[0]
    dst = edge_index[1]
    pad = ((0, 0), (0, EPTP - EPT))
    src3 = jnp.pad(src.reshape(NS, EPT), pad).reshape(NS, NCHUNK, K)
    dst3 = jnp.pad(dst.reshape(NS, EPT), pad).reshape(NS, NCHUNK, K)
    ew2 = jnp.pad(edge_weight.reshape(NS, EPT), pad)
    tsrc, tdst = _get_sc_bincount()(src, dst)
    ns = lax.rsqrt(jnp.maximum(jnp.sum(tsrc, axis=0)[:N], 1.0))[:, None]
    nd = lax.rsqrt(jnp.maximum(jnp.sum(tdst, axis=0)[:N], 1.0))[:, None]
    u1 = _mm1(updated_feats, ns, W1)
    p1 = _get_sc_scatter()(src3, dst3, ew2, u1)
    u2 = _mm2(p1, ns, nd, W2)
    p2 = _get_sc_scatter()(src3, dst3, ew2, u2)
    return _fin(p2, nd)


# speedup vs baseline: 1.2965x; 1.2965x over previous
import functools

import jax
import jax.numpy as jnp
import numpy as np
from jax import lax
from jax.experimental import pallas as pl
from jax.experimental.pallas import tpu as pltpu
from jax.experimental.pallas import tpu_sc as plsc

N = 10000
E = 320000
D = 128
D2 = D // 2
NEG_SLOPE = 0.01

NC = 2
NS = 16
L = 16
NW = NC * NS
K = 80
EPT = E // NS
NCHUNK = -(-EPT // K)
EPTP = NCHUNK * K
NP = 10240
RPT = NP // NS
ZB = 128


def _sc_mesh():
    return plsc.VectorSubcoreMesh(
        core_axis_name="c", subcore_axis_name="s", num_cores=NC, num_subcores=NS
    )


EPW = E // NW


@functools.cache
def _get_sc_bincount():
    @functools.partial(
        pl.kernel,
        out_type=(
            jax.ShapeDtypeStruct((NW, NP), jnp.float32),
            jax.ShapeDtypeStruct((NW, NP), jnp.float32),
        ),
        mesh=_sc_mesh(),
        compiler_params=pltpu.CompilerParams(needs_layout_passes=False),
        scratch_types=[
            pltpu.VMEM((EPW,), jnp.int32),
            pltpu.VMEM((EPW,), jnp.int32),
            pltpu.VMEM((NP,), jnp.float32),
            pltpu.VMEM((NP,), jnp.float32),
        ],
    )
    def _sc_bincount(src_h, dst_h, osrc_h, odst_h, sall, dall, csrc, cdst):
        c = lax.axis_index("c")
        s = lax.axis_index("s")
        wid = c * NS + s

        ebase = wid * EPW
        pltpu.sync_copy(src_h.at[pl.ds(ebase, EPW)], sall)
        pltpu.sync_copy(dst_h.at[pl.ds(ebase, EPW)], dall)

        def zero(i, _):
            z = jnp.zeros((L,), jnp.float32)
            csrc[pl.ds(i * L, L)] = z
            cdst[pl.ds(i * L, L)] = z
            return 0

        lax.fori_loop(0, NP // L, zero, 0)

        def grp(g, _):
            sv = sall[pl.ds(g * L, L)]
            cnt, last = plsc.scan_count(sv)
            plsc.addupdate_scatter(csrc, [sv], cnt.astype(jnp.float32), mask=last)
            dv = dall[pl.ds(g * L, L)]
            cnt2, last2 = plsc.scan_count(dv)
            plsc.addupdate_scatter(cdst, [dv], cnt2.astype(jnp.float32), mask=last2)
            return 0

        lax.fori_loop(0, EPW // L, grp, 0)

        pltpu.sync_copy(csrc, osrc_h.at[wid])
        pltpu.sync_copy(cdst, odst_h.at[wid])

    return _sc_bincount


@functools.cache
def _get_sc_scatter():
    @functools.partial(
        pl.kernel,
        out_type=jax.ShapeDtypeStruct((NC, NP, D2), jnp.float32),
        mesh=_sc_mesh(),
        compiler_params=pltpu.CompilerParams(
            needs_layout_passes=False, use_tc_tiling_on_sc=False
        ),
        scratch_types=[
            pltpu.VMEM((NCHUNK, K), jnp.int32),
            pltpu.VMEM((NCHUNK, K), jnp.int32),
            pltpu.VMEM((EPTP,), jnp.float32),
            pltpu.VMEM((2, K, D2), jnp.bfloat16),
            pltpu.VMEM((K, D2), jnp.float32),
            pltpu.VMEM((ZB, D2), jnp.float32),
            pltpu.VMEM_SHARED((NP, D2), jnp.float32),
            pltpu.SemaphoreType.DMA((2,)),
        ],
    )
    def _sc_scatter(src_h, dst_h, ew_h, u_h, out_h,
                    sidx, didx, ewv, rows, rows_f, zer, agg, sem):
        c = lax.axis_index("c")
        s = lax.axis_index("s")

        pltpu.sync_copy(src_h.at[s], sidx)
        pltpu.sync_copy(dst_h.at[s], didx)
        pltpu.sync_copy(ew_h.at[s], ewv)

        def fill_zer(i, _):
            for f in range(D2 // L):
                zer[i, pl.ds(f * L, L)] = jnp.zeros((L,), jnp.float32)
            return 0

        lax.fori_loop(0, ZB, fill_zer, 0)

        r0 = s * RPT
        for j in range(RPT // ZB):
            pltpu.sync_copy(zer, agg.at[pl.ds(r0 + j * ZB, ZB)])
        plsc.subcore_barrier()

        uc = u_h.at[c]

        def start_gather(i):
            b = lax.rem(i, 2)
            pltpu.async_copy(uc.at[sidx.at[i]], rows.at[b], sem.at[b])

        def wait_gather(i):
            b = lax.rem(i, 2)
            pltpu.make_async_copy(uc.at[sidx.at[i]], rows.at[b], sem.at[b]).wait()

        start_gather(0)

        def chunk(i, _):
            b = lax.rem(i, 2)
            wait_gather(i)

            @pl.when(i < NCHUNK - 1)
            def _():
                start_gather(i + 1)

            def scale(g, _):
                cvec = ewv[pl.ds(i * K + g * L, L)]
                for r in range(L):
                    ce = jnp.full((L,), cvec[r], jnp.float32)
                    e = g * L + r
                    for f in range(D2 // 32):
                        vb = rows[b, e, pl.ds(f * 32, 32)]
                        vi = plsc.bitcast(vb, jnp.int32)
                        lo = plsc.bitcast(vi << 16, jnp.float32)
                        hi = plsc.bitcast(vi & jnp.int32(-65536), jnp.float32)
                        rows_f[e, pl.ds(f * 32, L)] = lo * ce
                        rows_f[e, pl.ds(f * 32 + L, L)] = hi * ce
                return 0

            lax.fori_loop(0, K // L, scale, 0)
            pltpu.sync_copy(rows_f, agg.at[didx.at[i]], add=True)
            return 0

        lax.fori_loop(0, NCHUNK, chunk, 0)
        plsc.subcore_barrier()

        for j in range(RPT // ZB):
            sl = pl.ds(r0 + j * ZB, ZB)
            pltpu.sync_copy(agg.at[sl], out_h.at[c, sl])

    return _sc_scatter


BM = 1000


def _split(y, o_ref):
    yb = y.astype(jnp.bfloat16)
    o_ref[0] = yb[:, :D2]
    o_ref[1] = yb[:, D2:]


def _mm1_body(x_ref, ns_ref, w_ref, o_ref):
    y = jnp.dot(x_ref[:] * ns_ref[:], w_ref[:], preferred_element_type=jnp.float32)
    _split(y, o_ref)


def _mm2_body(p_ref, ns_ref, nd_ref, w_ref, o_ref):
    h = jnp.concatenate([p_ref[0], p_ref[1]], axis=1) * nd_ref[:]
    h = jnp.where(h > 0, h, NEG_SLOPE * h)
    y = jnp.dot(h * ns_ref[:], w_ref[:], preferred_element_type=jnp.float32)
    _split(y, o_ref)


def _fin_body(p_ref, nd_ref, o_ref):
    h = jnp.concatenate([p_ref[0], p_ref[1]], axis=1) * nd_ref[:]
    h = jnp.where(h > 0, h, NEG_SLOPE * h)
    part = jnp.sum(h, axis=0, keepdims=True) * (1.0 / N)

    @pl.when(pl.program_id(0) == 0)
    def _():
        o_ref[:] = jnp.zeros_like(o_ref)

    o_ref[:] = o_ref[:] + part


_col_spec = pl.BlockSpec((BM, 1), lambda i: (i, 0))
_p_spec = pl.BlockSpec((NC, BM, D2), lambda i: (0, i, 0))
_u_spec = pl.BlockSpec((NC, BM, D2), lambda i: (0, i, 0))
_w_spec = pl.BlockSpec((D, D), lambda i: (0, 0))

_mm1 = pl.pallas_call(
    _mm1_body,
    grid=(N // BM,),
    in_specs=[pl.BlockSpec((BM, D), lambda i: (i, 0)), _col_spec, _w_spec],
    out_specs=_u_spec,
    out_shape=jax.ShapeDtypeStruct((NC, N, D2), jnp.bfloat16),
)

_mm2 = pl.pallas_call(
    _mm2_body,
    grid=(N // BM,),
    in_specs=[_p_spec, _col_spec, _col_spec, _w_spec],
    out_specs=_u_spec,
    out_shape=jax.ShapeDtypeStruct((NC, N, D2), jnp.bfloat16),
)

_fin = pl.pallas_call(
    _fin_body,
    grid=(N // BM,),
    in_specs=[_p_spec, _col_spec],
    out_specs=pl.BlockSpec((1, D), lambda i: (0, 0)),
    out_shape=jax.ShapeDtypeStruct((1, D), jnp.float32),
)


_J16 = np.arange(16)
_GINV32 = np.empty(32, np.int32)
_GINV32[2 * _J16] = _J16
_GINV32[2 * _J16 + 1] = 16 + _J16
_GINV64 = np.concatenate([_GINV32, 32 + _GINV32])
_RHO = np.concatenate([_GINV64, 64 + _GINV64])


def kernel(updated_feats, edge_index, edge_weight, W1, W2):
    src = edge_index[0]
    dst = edge_index[1]
    pad = ((0, 0), (0, EPTP - EPT))
    src3 = jnp.pad(src.reshape(NS, EPT), pad).reshape(NS, NCHUNK, K)
    dst3 = jnp.pad(dst.reshape(NS, EPT), pad).reshape(NS, NCHUNK, K)
    ew2 = jnp.pad(edge_weight.reshape(NS, EPT), pad)
    W1p = W1[:, _RHO]
    W2p = W2[:, _RHO]
    tsrc, tdst = _get_sc_bincount()(src, dst)
    ns = lax.rsqrt(jnp.maximum(jnp.sum(tsrc, axis=0)[:N], 1.0))[:, None]
    nd = lax.rsqrt(jnp.maximum(jnp.sum(tdst, axis=0)[:N], 1.0))[:, None]
    u1 = _mm1(updated_feats, ns, W1p)
    p1 = _get_sc_scatter()(src3, dst3, ew2, u1)
    u2 = _mm2(p1, ns, nd, W2p)
    p2 = _get_sc_scatter()(src3, dst3, ew2, u2)
    return _fin(p2, nd)
